# baseline (speedup 1.0000x reference)
import dataclasses
import functools

import jax
import jax.numpy as jnp
from jax import lax
from jax.experimental import pallas as pl
from jax.experimental.pallas import tpu as pltpu
from jax.experimental.pallas import tpu_sc as plsc

HIDDEN = 768
NUM_EXPERTS = 8
TOP_K = 2
DFF = 1024
TOKENS = 2048
EPS = 1e-06

BT = 512
NB = 16
NS = NB * BT
NWORK = 32
SLOTS_PER_W = NS // NWORK
GCH = 64
NCH = SLOTS_PER_W // GCH


def _cumsum0(x):
    c = x
    s = 1
    while s < x.shape[0]:
        z = jnp.zeros((s, x.shape[1]), x.dtype)
        c = c + jnp.concatenate([z, c[:-s, :]], axis=0)
        s *= 2
    return c


def _lane_cumsum(x):
    c = x
    s = 1
    while s < x.shape[1]:
        z = jnp.zeros((x.shape[0], s), x.dtype)
        c = c + jnp.concatenate([z, c[:, :-s]], axis=1)
        s *= 2
    return c


def _router_body(rin_ref, rscale_ref, rproj_ref, pes_ref,
                 dest_ref, wts_ref, b2e_ref):
    x = rin_ref[...]
    var = jnp.mean(jnp.square(x), axis=-1, keepdims=True)
    x = x * lax.rsqrt(var + EPS)
    x = x * rscale_ref[...] * (HIDDEN ** -0.5)
    logits = jnp.dot(
        x.astype(jnp.bfloat16),
        rproj_ref[...].astype(jnp.bfloat16),
        preferred_element_type=jnp.float32,
    )
    probs = jax.nn.softmax(logits, axis=-1)

    iota = lax.broadcasted_iota(jnp.int32, probs.shape, 1)
    m1 = jnp.max(probs, axis=-1, keepdims=True)
    a1 = jnp.min(jnp.where(probs == m1, iota, NUM_EXPERTS), axis=-1,
                 keepdims=True)
    one1 = (iota == a1).astype(jnp.float32)
    probs2 = jnp.where(one1 > 0, -jnp.inf, probs)
    m2 = jnp.max(probs2, axis=-1, keepdims=True)
    a2 = jnp.min(jnp.where(probs2 == m2, iota, NUM_EXPERTS), axis=-1,
                 keepdims=True)
    one2 = (iota == a2).astype(jnp.float32)

    denom = m1 + m2 + 1e-20
    pes = pes_ref[...]
    w1t = (m1 / denom) * jnp.sum(one1 * pes, axis=-1, keepdims=True)
    w2t = (m2 / denom) * jnp.sum(one2 * pes, axis=-1, keepdims=True)

    ind = one1 + one2
    cum = _cumsum0(ind)
    rank_excl = cum - ind
    counts = cum[TOKENS - 1:TOKENS, :]
    counts_i = counts.astype(jnp.int32)
    padded = ((counts_i + (BT - 1)) // BT) * BT
    ends = _lane_cumsum(padded)
    offs = (ends - padded).astype(jnp.float32)

    slot = offs + rank_excl
    d1 = jnp.sum(one1 * slot, axis=-1, keepdims=True)
    d2 = jnp.sum(one2 * slot, axis=-1, keepdims=True)
    dest_ref[...] = jnp.concatenate([d1, d2], axis=1).astype(jnp.int32)
    wts_ref[...] = jnp.concatenate([w1t, w2t], axis=1)

    lane8 = lax.broadcasted_iota(jnp.int32, (1, NUM_EXPERTS), 1)
    last_e = jnp.max(jnp.where(padded > 0, lane8, 0))
    nact = ends[0, NUM_EXPERTS - 1] // BT
    biota = lax.broadcasted_iota(jnp.int32, (1, 32), 1)
    acc = jnp.zeros((1, 32), jnp.int32)
    for e in range(NUM_EXPERTS):
        acc = acc + (biota * BT >= ends[0, e]).astype(jnp.int32)
    b2e = jnp.minimum(acc, last_e)
    b2e_ref[...] = jnp.where(biota == NB, nact, b2e)


def _dispatch_body(dflat_ref, wflat_ref, x_ref, xs_ref, ws_ref,
                   dv, wv, sids, wsv, rows0, rows1, sem0, sem1):
    wid = lax.axis_index("s") * 2 + lax.axis_index("c")
    base = wid * SLOTS_PER_W
    pltpu.sync_copy(dflat_ref, dv)
    pltpu.sync_copy(wflat_ref, wv)

    zi = jnp.zeros((16,), jnp.int32)
    zf = jnp.zeros((16,), jnp.float32)

    @pl.loop(0, SLOTS_PER_W, step=16)
    def _(i):
        sids[pl.ds(i, 16)] = zi
        wsv[pl.ds(i, 16)] = zf

    i16 = lax.iota(jnp.int32, 16)

    @pl.loop(0, TOP_K * TOKENS, step=16)
    def _(i):
        idx = dv[pl.ds(i, 16)]
        rel = idx - base
        m = (rel >= 0) & (rel < SLOTS_PER_W)
        relc = jnp.where(m, rel, 0)
        tval = lax.shift_right_logical(i16 + i, 1)
        w = wv[pl.ds(i, 16)]
        plsc.store_scatter(sids, [relc], tval, mask=m)
        plsc.store_scatter(wsv, [relc], w, mask=m)

    bufs = (rows0, rows1)
    sems = (sem0, sem1)
    cps = [None] * NCH
    for c in range(min(2, NCH)):
        cps[c] = pltpu.async_copy(
            x_ref.at[sids.at[pl.ds(c * GCH, GCH)]], bufs[c % 2], sems[c % 2])
    for c in range(NCH):
        cps[c].wait()
        pltpu.sync_copy(bufs[c % 2], xs_ref.at[pl.ds(base + c * GCH, GCH)])
        nxt = c + 2
        if nxt < NCH:
            cps[nxt] = pltpu.async_copy(
                x_ref.at[sids.at[pl.ds(nxt * GCH, GCH)]],
                bufs[nxt % 2], sems[nxt % 2])

    pltpu.sync_copy(wsv, ws_ref.at[pl.ds(base, SLOTS_PER_W)])


def _expert_body(b2e_ref, xs_ref, ws_ref, w1_ref, w3_ref, w2_ref, ys_ref):
    b = pl.program_id(0)
    nact = b2e_ref[NB]

    @pl.when(b < nact)
    def _():
        x = xs_ref[...].astype(jnp.bfloat16)
        g = jnp.dot(x, w1_ref[0].astype(jnp.bfloat16),
                    preferred_element_type=jnp.float32).astype(jnp.bfloat16)
        u = jnp.dot(x, w3_ref[0].astype(jnp.bfloat16),
                    preferred_element_type=jnp.float32).astype(jnp.bfloat16)
        h = jax.nn.gelu(g) * u
        y = jnp.dot(h, w2_ref[0].astype(jnp.bfloat16),
                    preferred_element_type=jnp.float32)
        ys_ref[...] = y * ws_ref[0]


def _gather_body(ys_ref, dflat_ref, g_ref, didx, rows, sem):
    wid = lax.axis_index("s") * 2 + lax.axis_index("c")
    n = (TOP_K * TOKENS) // NWORK
    base = wid * n
    pltpu.sync_copy(dflat_ref.at[pl.ds(base, n)], didx)
    pltpu.async_copy(ys_ref.at[didx], rows, sem).wait()
    pltpu.sync_copy(rows, g_ref.at[pl.ds(base, n)])


def _combine_body(g2_ref, out_ref):
    out_ref[...] = g2_ref[:, :HIDDEN] + g2_ref[:, HIDDEN:]


@jax.jit
def kernel(hidden_states, router_input, router_scale, router_proj,
           per_expert_scale, w1, w2, w3):
    T, H = hidden_states.shape
    E = NUM_EXPERTS

    dest2, wts2, b2e = pl.pallas_call(
        _router_body,
        out_shape=[
            jax.ShapeDtypeStruct((T, TOP_K), jnp.int32),
            jax.ShapeDtypeStruct((T, TOP_K), jnp.float32),
            jax.ShapeDtypeStruct((1, 32), jnp.int32),
        ],
        in_specs=[
            pl.BlockSpec((T, H), lambda: (0, 0)),
            pl.BlockSpec((1, H), lambda: (0, 0)),
            pl.BlockSpec((H, E), lambda: (0, 0)),
            pl.BlockSpec((1, E), lambda: (0, 0)),
        ],
        out_specs=[
            pl.BlockSpec((T, TOP_K), lambda: (0, 0)),
            pl.BlockSpec((T, TOP_K), lambda: (0, 0)),
            pl.BlockSpec((1, 32), lambda: (0, 0)),
        ],
    )(router_input, router_scale.reshape(1, H), router_proj,
      per_expert_scale.reshape(1, E))

    dflat = dest2.reshape(TOP_K * T)
    wflat = wts2.reshape(TOP_K * T)

    mesh = plsc.VectorSubcoreMesh(core_axis_name="c", subcore_axis_name="s")
    sc_params = pltpu.CompilerParams()
    if "needs_layout_passes" in pltpu.CompilerParams.__dataclass_fields__:
        sc_params = dataclasses.replace(sc_params, needs_layout_passes=False)

    @functools.partial(
        pl.kernel,
        mesh=mesh,
        out_type=[
            jax.ShapeDtypeStruct((NS, H), jnp.float32),
            jax.ShapeDtypeStruct((NS,), jnp.float32),
        ],
        scratch_types=[
            pltpu.VMEM((TOP_K * T,), jnp.int32),
            pltpu.VMEM((TOP_K * T,), jnp.float32),
            pltpu.VMEM((SLOTS_PER_W,), jnp.int32),
            pltpu.VMEM((SLOTS_PER_W,), jnp.float32),
            pltpu.VMEM((GCH, H), jnp.float32),
            pltpu.VMEM((GCH, H), jnp.float32),
            pltpu.SemaphoreType.DMA,
            pltpu.SemaphoreType.DMA,
        ],
        compiler_params=sc_params,
    )
    def _dispatch(dflat_ref, wflat_ref, x_ref, xs_ref, ws_ref,
                  dv, wv, sids, wsv, rows0, rows1, sem0, sem1):
        _dispatch_body(dflat_ref, wflat_ref, x_ref, xs_ref, ws_ref,
                       dv, wv, sids, wsv, rows0, rows1, sem0, sem1)

    xs, ws = _dispatch(dflat, wflat, hidden_states)
    ws3 = ws.reshape(NB, BT, 1)

    ys = pl.pallas_call(
        _expert_body,
        grid_spec=pltpu.PrefetchScalarGridSpec(
            num_scalar_prefetch=1,
            grid=(NB,),
            in_specs=[
                pl.BlockSpec((BT, H), lambda b, b2e: (b, 0)),
                pl.BlockSpec((1, BT, 1), lambda b, b2e: (b, 0, 0)),
                pl.BlockSpec((1, H, DFF), lambda b, b2e: (b2e[b], 0, 0)),
                pl.BlockSpec((1, H, DFF), lambda b, b2e: (b2e[b], 0, 0)),
                pl.BlockSpec((1, DFF, H), lambda b, b2e: (b2e[b], 0, 0)),
            ],
            out_specs=pl.BlockSpec((BT, H), lambda b, b2e: (b, 0)),
        ),
        out_shape=jax.ShapeDtypeStruct((NS, H), jnp.float32),
        compiler_params=pltpu.CompilerParams(
            dimension_semantics=("arbitrary",),
        ),
    )(b2e.reshape(32), xs, ws3, w1, w3, w2)

    @functools.partial(
        pl.kernel,
        mesh=mesh,
        out_type=jax.ShapeDtypeStruct((TOP_K * T, H), jnp.float32),
        scratch_types=[
            pltpu.VMEM(((TOP_K * T) // NWORK,), jnp.int32),
            pltpu.VMEM(((TOP_K * T) // NWORK, H), jnp.float32),
            pltpu.SemaphoreType.DMA,
        ],
        compiler_params=sc_params,
    )
    def _gather(ys_ref, dflat_ref, g_ref, didx, rows, sem):
        _gather_body(ys_ref, dflat_ref, g_ref, didx, rows, sem)

    g = _gather(ys, dflat)
    g2 = g.reshape(T, TOP_K * H)

    out = pl.pallas_call(
        _combine_body,
        grid=(4,),
        in_specs=[pl.BlockSpec((T // 4, TOP_K * H), lambda i: (i, 0))],
        out_specs=pl.BlockSpec((T // 4, H), lambda i: (i, 0)),
        out_shape=jax.ShapeDtypeStruct((T, H), jnp.float32),
    )(g2)
    return out


# speedup vs baseline: 1.3276x; 1.3276x over previous
import dataclasses
import functools

import jax
import jax.numpy as jnp
from jax import lax
from jax.experimental import pallas as pl
from jax.experimental.pallas import tpu as pltpu
from jax.experimental.pallas import tpu_sc as plsc

HIDDEN = 768
NUM_EXPERTS = 8
TOP_K = 2
DFF = 1024
TOKENS = 2048
EPS = 1e-06

BT = 256
NB = 24
NS = NB * BT
NWORK = 32
SLOTS_PER_W = NS // NWORK
GCH = 48
NCH = SLOTS_PER_W // GCH


def _cumsum0(x):
    c = x
    s = 1
    while s < x.shape[0]:
        z = jnp.zeros((s, x.shape[1]), x.dtype)
        c = c + jnp.concatenate([z, c[:-s, :]], axis=0)
        s *= 2
    return c


def _lane_cumsum(x):
    c = x
    s = 1
    while s < x.shape[1]:
        z = jnp.zeros((x.shape[0], s), x.dtype)
        c = c + jnp.concatenate([z, c[:, :-s]], axis=1)
        s *= 2
    return c


def _router_body(rin_ref, rscale_ref, rproj_ref, pes_ref,
                 dest_ref, wts_ref, b2e_ref):
    x = rin_ref[...]
    var = jnp.mean(jnp.square(x), axis=-1, keepdims=True)
    x = x * lax.rsqrt(var + EPS)
    x = x * rscale_ref[...] * (HIDDEN ** -0.5)
    logits = jnp.dot(
        x.astype(jnp.bfloat16),
        rproj_ref[...].astype(jnp.bfloat16),
        preferred_element_type=jnp.float32,
    )
    probs = jax.nn.softmax(logits, axis=-1)

    iota = lax.broadcasted_iota(jnp.int32, probs.shape, 1)
    m1 = jnp.max(probs, axis=-1, keepdims=True)
    a1 = jnp.min(jnp.where(probs == m1, iota, NUM_EXPERTS), axis=-1,
                 keepdims=True)
    one1 = (iota == a1).astype(jnp.float32)
    probs2 = jnp.where(one1 > 0, -jnp.inf, probs)
    m2 = jnp.max(probs2, axis=-1, keepdims=True)
    a2 = jnp.min(jnp.where(probs2 == m2, iota, NUM_EXPERTS), axis=-1,
                 keepdims=True)
    one2 = (iota == a2).astype(jnp.float32)

    denom = m1 + m2 + 1e-20
    pes = pes_ref[...]
    w1t = (m1 / denom) * jnp.sum(one1 * pes, axis=-1, keepdims=True)
    w2t = (m2 / denom) * jnp.sum(one2 * pes, axis=-1, keepdims=True)

    ind = one1 + one2
    cum = _cumsum0(ind)
    rank_excl = cum - ind
    counts = cum[TOKENS - 1:TOKENS, :]
    counts_i = counts.astype(jnp.int32)
    padded = ((counts_i + (BT - 1)) // BT) * BT
    ends = _lane_cumsum(padded)
    offs = (ends - padded).astype(jnp.float32)

    slot = offs + rank_excl
    d1 = jnp.sum(one1 * slot, axis=-1, keepdims=True)
    d2 = jnp.sum(one2 * slot, axis=-1, keepdims=True)
    dest_ref[...] = jnp.concatenate([d1, d2], axis=1).astype(jnp.int32)
    wts_ref[...] = jnp.concatenate([w1t, w2t], axis=1)

    lane8 = lax.broadcasted_iota(jnp.int32, (1, NUM_EXPERTS), 1)
    last_e = jnp.max(jnp.where(padded > 0, lane8, 0))
    nact = ends[0, NUM_EXPERTS - 1] // BT
    biota = lax.broadcasted_iota(jnp.int32, (1, 32), 1)
    acc = jnp.zeros((1, 32), jnp.int32)
    for e in range(NUM_EXPERTS):
        acc = acc + (biota * BT >= ends[0, e]).astype(jnp.int32)
    b2e = jnp.minimum(acc, last_e)
    b2e_ref[...] = jnp.where(biota == NB, nact, b2e)


def _dispatch_body(dflat_ref, wflat_ref, x_ref, xs_ref, ws_ref,
                   dv, wv, sids, wsv, rows0, rows1, sem0, sem1):
    wid = lax.axis_index("s") * 2 + lax.axis_index("c")
    base = wid * SLOTS_PER_W
    pltpu.sync_copy(dflat_ref, dv)
    pltpu.sync_copy(wflat_ref, wv)

    zi = jnp.zeros((16,), jnp.int32)
    zf = jnp.zeros((16,), jnp.float32)

    @pl.loop(0, SLOTS_PER_W, step=16)
    def _(i):
        sids[pl.ds(i, 16)] = zi
        wsv[pl.ds(i, 16)] = zf

    i16 = lax.iota(jnp.int32, 16)

    @pl.loop(0, TOP_K * TOKENS, step=16)
    def _(i):
        idx = dv[pl.ds(i, 16)]
        rel = idx - base
        m = (rel >= 0) & (rel < SLOTS_PER_W)
        relc = jnp.where(m, rel, 0)
        tval = lax.shift_right_logical(i16 + i, 1)
        w = wv[pl.ds(i, 16)]
        plsc.store_scatter(sids, [relc], tval, mask=m)
        plsc.store_scatter(wsv, [relc], w, mask=m)

    bufs = (rows0, rows1)
    sems = (sem0, sem1)
    cps = [None] * NCH
    for c in range(min(2, NCH)):
        cps[c] = pltpu.async_copy(
            x_ref.at[sids.at[pl.ds(c * GCH, GCH)]], bufs[c % 2], sems[c % 2])
    for c in range(NCH):
        cps[c].wait()
        pltpu.sync_copy(bufs[c % 2], xs_ref.at[pl.ds(base + c * GCH, GCH)])
        nxt = c + 2
        if nxt < NCH:
            cps[nxt] = pltpu.async_copy(
                x_ref.at[sids.at[pl.ds(nxt * GCH, GCH)]],
                bufs[nxt % 2], sems[nxt % 2])

    pltpu.sync_copy(wsv, ws_ref.at[pl.ds(base, SLOTS_PER_W)])


def _expert_body(b2e_ref, xs_ref, ws_ref, w1_ref, w3_ref, w2_ref, ys_ref):
    b = pl.program_id(0)
    nact = b2e_ref[NB]

    @pl.when(b < nact)
    def _():
        x = xs_ref[...].astype(jnp.bfloat16)
        g = jnp.dot(x, w1_ref[0].astype(jnp.bfloat16),
                    preferred_element_type=jnp.float32).astype(jnp.bfloat16)
        u = jnp.dot(x, w3_ref[0].astype(jnp.bfloat16),
                    preferred_element_type=jnp.float32).astype(jnp.bfloat16)
        h = jax.nn.gelu(g) * u
        y = jnp.dot(h, w2_ref[0].astype(jnp.bfloat16),
                    preferred_element_type=jnp.float32)
        ys_ref[...] = y * ws_ref[0]


def _gather_body(ys_ref, dflat_ref, g_ref, didx, rows, sem):
    wid = lax.axis_index("s") * 2 + lax.axis_index("c")
    n = (TOP_K * TOKENS) // NWORK
    base = wid * n
    pltpu.sync_copy(dflat_ref.at[pl.ds(base, n)], didx)
    pltpu.async_copy(ys_ref.at[didx], rows, sem).wait()
    pltpu.sync_copy(rows, g_ref.at[pl.ds(base, n)])


def _combine_body(g2_ref, out_ref):
    out_ref[...] = g2_ref[:, :HIDDEN] + g2_ref[:, HIDDEN:]


@jax.jit
def kernel(hidden_states, router_input, router_scale, router_proj,
           per_expert_scale, w1, w2, w3):
    T, H = hidden_states.shape
    E = NUM_EXPERTS

    dest2, wts2, b2e = pl.pallas_call(
        _router_body,
        out_shape=[
            jax.ShapeDtypeStruct((T, TOP_K), jnp.int32),
            jax.ShapeDtypeStruct((T, TOP_K), jnp.float32),
            jax.ShapeDtypeStruct((1, 32), jnp.int32),
        ],
        in_specs=[
            pl.BlockSpec((T, H), lambda: (0, 0)),
            pl.BlockSpec((1, H), lambda: (0, 0)),
            pl.BlockSpec((H, E), lambda: (0, 0)),
            pl.BlockSpec((1, E), lambda: (0, 0)),
        ],
        out_specs=[
            pl.BlockSpec((T, TOP_K), lambda: (0, 0)),
            pl.BlockSpec((T, TOP_K), lambda: (0, 0)),
            pl.BlockSpec((1, 32), lambda: (0, 0)),
        ],
    )(router_input, router_scale.reshape(1, H), router_proj,
      per_expert_scale.reshape(1, E))

    dflat = dest2.reshape(TOP_K * T)
    wflat = wts2.reshape(TOP_K * T)

    mesh = plsc.VectorSubcoreMesh(core_axis_name="c", subcore_axis_name="s")
    sc_params = pltpu.CompilerParams()
    if "needs_layout_passes" in pltpu.CompilerParams.__dataclass_fields__:
        sc_params = dataclasses.replace(sc_params, needs_layout_passes=False)

    @functools.partial(
        pl.kernel,
        mesh=mesh,
        out_type=[
            jax.ShapeDtypeStruct((NS, H), jnp.float32),
            jax.ShapeDtypeStruct((NS,), jnp.float32),
        ],
        scratch_types=[
            pltpu.VMEM((TOP_K * T,), jnp.int32),
            pltpu.VMEM((TOP_K * T,), jnp.float32),
            pltpu.VMEM((SLOTS_PER_W,), jnp.int32),
            pltpu.VMEM((SLOTS_PER_W,), jnp.float32),
            pltpu.VMEM((GCH, H), jnp.float32),
            pltpu.VMEM((GCH, H), jnp.float32),
            pltpu.SemaphoreType.DMA,
            pltpu.SemaphoreType.DMA,
        ],
        compiler_params=sc_params,
    )
    def _dispatch(dflat_ref, wflat_ref, x_ref, xs_ref, ws_ref,
                  dv, wv, sids, wsv, rows0, rows1, sem0, sem1):
        _dispatch_body(dflat_ref, wflat_ref, x_ref, xs_ref, ws_ref,
                       dv, wv, sids, wsv, rows0, rows1, sem0, sem1)

    xs, ws = _dispatch(dflat, wflat, hidden_states)
    ws3 = ws.reshape(NB, BT, 1)

    ys = pl.pallas_call(
        _expert_body,
        grid_spec=pltpu.PrefetchScalarGridSpec(
            num_scalar_prefetch=1,
            grid=(NB,),
            in_specs=[
                pl.BlockSpec((BT, H), lambda b, b2e: (b, 0)),
                pl.BlockSpec((1, BT, 1), lambda b, b2e: (b, 0, 0)),
                pl.BlockSpec((1, H, DFF), lambda b, b2e: (b2e[b], 0, 0)),
                pl.BlockSpec((1, H, DFF), lambda b, b2e: (b2e[b], 0, 0)),
                pl.BlockSpec((1, DFF, H), lambda b, b2e: (b2e[b], 0, 0)),
            ],
            out_specs=pl.BlockSpec((BT, H), lambda b, b2e: (b, 0)),
        ),
        out_shape=jax.ShapeDtypeStruct((NS, H), jnp.float32),
        compiler_params=pltpu.CompilerParams(
            dimension_semantics=("arbitrary",),
        ),
    )(b2e.reshape(32), xs, ws3, w1, w3, w2)

    @functools.partial(
        pl.kernel,
        mesh=mesh,
        out_type=jax.ShapeDtypeStruct((TOP_K * T, H), jnp.float32),
        scratch_types=[
            pltpu.VMEM(((TOP_K * T) // NWORK,), jnp.int32),
            pltpu.VMEM(((TOP_K * T) // NWORK, H), jnp.float32),
            pltpu.SemaphoreType.DMA,
        ],
        compiler_params=sc_params,
    )
    def _gather(ys_ref, dflat_ref, g_ref, didx, rows, sem):
        _gather_body(ys_ref, dflat_ref, g_ref, didx, rows, sem)

    g = _gather(ys, dflat)
    g2 = g.reshape(T, TOP_K * H)

    out = pl.pallas_call(
        _combine_body,
        grid=(4,),
        in_specs=[pl.BlockSpec((T // 4, TOP_K * H), lambda i: (i, 0))],
        out_specs=pl.BlockSpec((T // 4, H), lambda i: (i, 0)),
        out_shape=jax.ShapeDtypeStruct((T, H), jnp.float32),
    )(g2)
    return out


# speedup vs baseline: 3.1982x; 2.4091x over previous
import jax
import jax.numpy as jnp
from jax import lax
from jax.experimental import pallas as pl
from jax.experimental.pallas import tpu as pltpu

HIDDEN = 768
NUM_EXPERTS = 8
TOP_K = 2
DFF = 1024
TOKENS = 2048
EPS = 1e-06


def _moe_body(hid_ref, rin_ref, rscale_ref, rproj_ref, pes_ref,
              w1_ref, w3_ref, w2_ref, out_ref, comb_ref, xbf_ref):
    e = pl.program_id(0)

    @pl.when(e == 0)
    def _():
        xbf_ref[...] = hid_ref[...].astype(jnp.bfloat16)
        x = rin_ref[...]
        var = jnp.mean(jnp.square(x), axis=-1, keepdims=True)
        x = x * lax.rsqrt(var + EPS)
        x = x * rscale_ref[...] * (HIDDEN ** -0.5)
        logits = jnp.dot(
            x.astype(jnp.bfloat16),
            rproj_ref[...].astype(jnp.bfloat16),
            preferred_element_type=jnp.float32,
        )
        probs = jax.nn.softmax(logits, axis=-1)

        iota = lax.broadcasted_iota(jnp.int32, probs.shape, 1)
        m1 = jnp.max(probs, axis=-1, keepdims=True)
        a1 = jnp.min(jnp.where(probs == m1, iota, NUM_EXPERTS), axis=-1,
                     keepdims=True)
        one1 = (iota == a1)
        probs2 = jnp.where(one1, -jnp.inf, probs)
        m2 = jnp.max(probs2, axis=-1, keepdims=True)
        a2 = jnp.min(jnp.where(probs2 == m2, iota, NUM_EXPERTS), axis=-1,
                     keepdims=True)
        one2 = (iota == a2)

        denom = m1 + m2 + 1e-20
        comb = (m1 * one1 + m2 * one2) / denom
        comb_ref[...] = comb * pes_ref[...]

    x = xbf_ref[...]
    g = jnp.dot(x, w1_ref[0].astype(jnp.bfloat16),
                preferred_element_type=jnp.float32).astype(jnp.bfloat16)
    u = jnp.dot(x, w3_ref[0].astype(jnp.bfloat16),
                preferred_element_type=jnp.float32).astype(jnp.bfloat16)
    h = jax.nn.gelu(g) * u
    y = jnp.dot(h, w2_ref[0].astype(jnp.bfloat16),
                preferred_element_type=jnp.float32)
    lane = lax.broadcasted_iota(jnp.int32, (1, NUM_EXPERTS), 1)
    c = jnp.sum(comb_ref[...] * (lane == e).astype(jnp.float32), axis=-1,
                keepdims=True)
    contrib = c * y

    @pl.when(e == 0)
    def _():
        out_ref[...] = contrib

    @pl.when(e != 0)
    def _():
        out_ref[...] += contrib


@jax.jit
def kernel(hidden_states, router_input, router_scale, router_proj,
           per_expert_scale, w1, w2, w3):
    T, H = hidden_states.shape
    E = NUM_EXPERTS

    out = pl.pallas_call(
        _moe_body,
        grid=(E,),
        out_shape=jax.ShapeDtypeStruct((T, H), jnp.float32),
        in_specs=[
            pl.BlockSpec((T, H), lambda e: (0, 0)),
            pl.BlockSpec((T, H), lambda e: (0, 0)),
            pl.BlockSpec((1, H), lambda e: (0, 0)),
            pl.BlockSpec((H, E), lambda e: (0, 0)),
            pl.BlockSpec((1, E), lambda e: (0, 0)),
            pl.BlockSpec((1, H, DFF), lambda e: (e, 0, 0)),
            pl.BlockSpec((1, H, DFF), lambda e: (e, 0, 0)),
            pl.BlockSpec((1, DFF, H), lambda e: (e, 0, 0)),
        ],
        out_specs=pl.BlockSpec((T, H), lambda e: (0, 0)),
        scratch_shapes=[
            pltpu.VMEM((T, E), jnp.float32),
            pltpu.VMEM((T, H), jnp.bfloat16),
        ],
        compiler_params=pltpu.CompilerParams(
            dimension_semantics=("arbitrary",),
        ),
    )(hidden_states, router_input, router_scale.reshape(1, H), router_proj,
      per_expert_scale.reshape(1, E), w1, w3, w2)
    return out
